# baseline (speedup 1.0000x reference)
import jax
import jax.numpy as jnp
from jax.experimental import pallas as pl
from jax.experimental.pallas import tpu as pltpu

CHUNK = 256
NBUF = 8
EPAD = 128
NEG = -1e9


def _router_kernel(x_hbm, w1_ref, b1_ref, w2_ref, b2_ref,
                   prob_ref, logit_ref, *scratch):
    bufs = scratch[:NBUF]
    sems = scratch[NBUF:]
    n_chunks = x_hbm.shape[0] // CHUNK
    w1 = w1_ref[...].astype(jnp.bfloat16)
    w2 = w2_ref[...].astype(jnp.bfloat16)

    def copy_in(i, slot):
        return pltpu.make_async_copy(
            x_hbm.at[pl.ds(i * CHUNK, CHUNK), :],
            bufs[slot],
            sems[slot],
        )

    for i in range(min(NBUF, n_chunks)):
        copy_in(i, i).start()

    for i in range(n_chunks):
        slot = i % NBUF
        copy_in(i, slot).wait()
        h = jax.nn.sigmoid(
            jnp.dot(bufs[slot][...].astype(jnp.bfloat16), w1,
                    preferred_element_type=jnp.float32)
            + b1_ref[...]
        )
        nxt = i + NBUF
        if nxt < n_chunks:
            copy_in(nxt, slot).start()
        logits = (
            jnp.dot(h.astype(jnp.bfloat16), w2,
                    preferred_element_type=jnp.float32)
            + b2_ref[...]
        )
        sl = pl.ds(i * CHUNK, CHUNK)
        logit_ref[sl, :] = logits
        m = jnp.max(logits, axis=1, keepdims=True)
        e = jnp.exp(logits - m)
        prob_ref[sl, :] = e / jnp.sum(e, axis=1, keepdims=True)


@jax.jit
def kernel(x, W1, b1, W2, b2):
    B, D = x.shape
    H = W1.shape[1]
    E = W2.shape[1]
    b1 = b1.reshape(1, H)
    W2p = jnp.pad(W2, ((0, 0), (0, EPAD - E)))
    b2p = jnp.pad(b2.reshape(1, E), ((0, 0), (0, EPAD - E)),
                  constant_values=NEG)
    probs, logits = pl.pallas_call(
        _router_kernel,
        in_specs=[
            pl.BlockSpec(memory_space=pl.ANY),
            pl.BlockSpec(memory_space=pltpu.VMEM),
            pl.BlockSpec(memory_space=pltpu.VMEM),
            pl.BlockSpec(memory_space=pltpu.VMEM),
            pl.BlockSpec(memory_space=pltpu.VMEM),
        ],
        out_specs=[
            pl.BlockSpec(memory_space=pltpu.VMEM),
            pl.BlockSpec(memory_space=pltpu.VMEM),
        ],
        out_shape=[
            jax.ShapeDtypeStruct((B, EPAD), jnp.float32),
            jax.ShapeDtypeStruct((B, EPAD), jnp.float32),
        ],
        scratch_shapes=(
            [pltpu.VMEM((CHUNK, D), jnp.float32) for _ in range(NBUF)]
            + [pltpu.SemaphoreType.DMA for _ in range(NBUF)]
        ),
    )(x, W1, b1, W2p, b2p)
    return (probs[:, :E], logits[:, :E])


# speedup vs baseline: 1.3687x; 1.3687x over previous
import jax
import jax.numpy as jnp
from jax.experimental import pallas as pl
from jax.experimental.pallas import tpu as pltpu

BLOCK_B = 1024


def _router_kernel(x_ref, w1_ref, b1_ref, w2_ref, b2_ref, prob_ref, logit_ref):
    h = jax.nn.sigmoid(
        jnp.dot(
            x_ref[...].astype(jnp.bfloat16),
            w1_ref[...].astype(jnp.bfloat16),
            preferred_element_type=jnp.float32,
        )
        + b1_ref[...]
    )
    logits = (
        jnp.dot(
            h.astype(jnp.bfloat16),
            w2_ref[...].astype(jnp.bfloat16),
            preferred_element_type=jnp.float32,
        )
        + b2_ref[...]
    )
    logit_ref[...] = logits
    m = jnp.max(logits, axis=1, keepdims=True)
    e = jnp.exp(logits - m)
    prob_ref[...] = e / jnp.sum(e, axis=1, keepdims=True)


@jax.jit
def kernel(x, W1, b1, W2, b2):
    B, D = x.shape
    H = W1.shape[1]
    E = W2.shape[1]
    b1 = b1.reshape(1, H)
    b2 = b2.reshape(1, E)
    grid = (B // BLOCK_B,)
    probs, logits = pl.pallas_call(
        _router_kernel,
        grid=grid,
        in_specs=[
            pl.BlockSpec((BLOCK_B, D), lambda i: (i, 0)),
            pl.BlockSpec((D, H), lambda i: (0, 0)),
            pl.BlockSpec((1, H), lambda i: (0, 0)),
            pl.BlockSpec((H, E), lambda i: (0, 0)),
            pl.BlockSpec((1, E), lambda i: (0, 0)),
        ],
        out_specs=[
            pl.BlockSpec((BLOCK_B, E), lambda i: (i, 0)),
            pl.BlockSpec((BLOCK_B, E), lambda i: (i, 0)),
        ],
        out_shape=[
            jax.ShapeDtypeStruct((B, E), jnp.float32),
            jax.ShapeDtypeStruct((B, E), jnp.float32),
        ],
        compiler_params=pltpu.CompilerParams(
            dimension_semantics=("arbitrary",),
        ),
    )(x, W1, b1, W2, b2)
    return (probs, logits)
